# baseline (speedup 1.0000x reference)
import jax
import jax.numpy as jnp
from jax.experimental import pallas as pl
from jax.experimental.pallas import tpu as pltpu

_GRID = 256
_VOX = 0.8
_NBOX = 20


def _loss_kernel(boxes_ref, a_ref, o_ref, inter_ref, union_ref, acc_a, acc_o):
    c = pl.program_id(1)
    nc = pl.num_programs(1)

    @pl.when(c == 0)
    def _():
        acc_a[...] = a_ref[0, 0]
        acc_o[...] = o_ref[0, 0]

    @pl.when(c > 0)
    def _():
        acc_a[...] += a_ref[0, 0]
        acc_o[...] += o_ref[0, 0]

    @pl.when(c == nc - 1)
    def _():
        pred_occ = acc_a[...] != 0.0
        orig_occ = acc_o[...] != 0.0
        ii = jax.lax.broadcasted_iota(jnp.int32, (_GRID, _GRID), 0)
        jj = jax.lax.broadcasted_iota(jnp.int32, (_GRID, _GRID), 1)
        x = (ii.astype(jnp.float32) - _GRID / 2.0) * _VOX
        y = (jj.astype(jnp.float32) - _GRID / 2.0) * _VOX
        boxes = boxes_ref[0]
        mask = jnp.zeros((_GRID, _GRID), dtype=jnp.bool_)
        for t in range(_NBOX):
            cx = boxes[t, 0]
            cy = boxes[t, 1]
            cz = boxes[t, 2]
            dx = boxes[t, 3]
            dy = boxes[t, 4]
            dz = boxes[t, 5]
            hd = boxes[t, 6]
            sx = x - cx
            sy = y - cy
            cth = jnp.cos(hd)
            sth = jnp.sin(hd)
            lx = sx * cth + sy * sth
            ly = sy * cth - sx * sth
            zin = jnp.abs(_VOX - cz) <= dz * 0.5
            inb = (jnp.abs(lx) <= dx * 0.5) & (jnp.abs(ly) <= dy * 0.5) & zin
            mask = mask | inb
        p = pred_occ & mask
        o = orig_occ & mask
        inter = jnp.sum(jnp.where(p & o, 1.0, 0.0))
        union = jnp.sum(jnp.where(p | o, 1.0, 0.0))
        inter_ref[0] = jnp.full((8, 128), inter, jnp.float32)
        union_ref[0] = jnp.full((8, 128), union, jnp.float32)


def kernel(added_points, original_points, boxes):
    B, C, H, W = added_points.shape
    boxes_p = jnp.zeros((B, 24, 128), jnp.float32).at[:, :_NBOX, :7].set(boxes)
    inter, union = pl.pallas_call(
        _loss_kernel,
        grid=(B, C),
        in_specs=[
            pl.BlockSpec((1, 24, 128), lambda b, c: (b, 0, 0)),
            pl.BlockSpec((1, 1, H, W), lambda b, c: (b, c, 0, 0)),
            pl.BlockSpec((1, 1, H, W), lambda b, c: (b, c + 1, 0, 0)),
        ],
        out_specs=[
            pl.BlockSpec((1, 8, 128), lambda b, c: (b, 0, 0)),
            pl.BlockSpec((1, 8, 128), lambda b, c: (b, 0, 0)),
        ],
        out_shape=[
            jax.ShapeDtypeStruct((B, 8, 128), jnp.float32),
            jax.ShapeDtypeStruct((B, 8, 128), jnp.float32),
        ],
        scratch_shapes=[
            pltpu.VMEM((H, W), jnp.float32),
            pltpu.VMEM((H, W), jnp.float32),
        ],
    )(boxes_p, added_points, original_points)
    iou = inter[:, 0, 0] / jnp.maximum(union[:, 0, 0], 1.0)
    return jnp.mean(iou)


# speedup vs baseline: 3.1411x; 3.1411x over previous
import jax
import jax.numpy as jnp
from jax.experimental import pallas as pl
from jax.experimental.pallas import tpu as pltpu

_GRID = 256
_VOX = 0.8
_NBOX = 20
_CHUNK = 16


def _loss_kernel(boxes_ref, a_ref, o_ref, o_last_ref, inter_ref, union_ref,
                 acc_a, acc_o):
    k = pl.program_id(1)
    nk = pl.num_programs(1)

    a_sum = jnp.sum(a_ref[0], axis=0)

    @pl.when(k == 0)
    def _():
        acc_a[...] = a_sum
        acc_o[...] = jnp.sum(o_ref[0, 1:], axis=0)

    @pl.when(k > 0)
    def _():
        acc_a[...] += a_sum
        acc_o[...] += jnp.sum(o_ref[0], axis=0)

    @pl.when(k == nk - 1)
    def _():
        pred_occ = acc_a[...] != 0.0
        orig_occ = (acc_o[...] + o_last_ref[0, 0]) != 0.0
        ii = jax.lax.broadcasted_iota(jnp.int32, (_GRID, _GRID), 0)
        jj = jax.lax.broadcasted_iota(jnp.int32, (_GRID, _GRID), 1)
        x = (ii.astype(jnp.float32) - _GRID / 2.0) * _VOX
        y = (jj.astype(jnp.float32) - _GRID / 2.0) * _VOX
        boxes = boxes_ref[0]
        mask = jnp.zeros((_GRID, _GRID), dtype=jnp.bool_)
        for t in range(_NBOX):
            cx = boxes[t, 0]
            cy = boxes[t, 1]
            cz = boxes[t, 2]
            dx = boxes[t, 3]
            dy = boxes[t, 4]
            dz = boxes[t, 5]
            hd = boxes[t, 6]
            sx = x - cx
            sy = y - cy
            cth = jnp.cos(hd)
            sth = jnp.sin(hd)
            lx = sx * cth + sy * sth
            ly = sy * cth - sx * sth
            zin = jnp.abs(_VOX - cz) <= dz * 0.5
            inb = (jnp.abs(lx) <= dx * 0.5) & (jnp.abs(ly) <= dy * 0.5) & zin
            mask = mask | inb
        p = pred_occ & mask
        o = orig_occ & mask
        inter = jnp.sum(jnp.where(p & o, 1.0, 0.0))
        union = jnp.sum(jnp.where(p | o, 1.0, 0.0))
        inter_ref[0] = jnp.full((8, 128), inter, jnp.float32)
        union_ref[0] = jnp.full((8, 128), union, jnp.float32)


def kernel(added_points, original_points, boxes):
    B, C, H, W = added_points.shape
    boxes_p = jnp.zeros((B, 24, 128), jnp.float32).at[:, :_NBOX, :7].set(boxes)
    nk = C // _CHUNK
    inter, union = pl.pallas_call(
        _loss_kernel,
        grid=(B, nk),
        in_specs=[
            pl.BlockSpec((1, 24, 128), lambda b, k: (b, 0, 0)),
            pl.BlockSpec((1, _CHUNK, H, W), lambda b, k: (b, k, 0, 0)),
            pl.BlockSpec((1, _CHUNK, H, W), lambda b, k: (b, k, 0, 0)),
            pl.BlockSpec((1, 1, H, W), lambda b, k: (b, C, 0, 0)),
        ],
        out_specs=[
            pl.BlockSpec((1, 8, 128), lambda b, k: (b, 0, 0)),
            pl.BlockSpec((1, 8, 128), lambda b, k: (b, 0, 0)),
        ],
        out_shape=[
            jax.ShapeDtypeStruct((B, 8, 128), jnp.float32),
            jax.ShapeDtypeStruct((B, 8, 128), jnp.float32),
        ],
        scratch_shapes=[
            pltpu.VMEM((H, W), jnp.float32),
            pltpu.VMEM((H, W), jnp.float32),
        ],
    )(boxes_p, added_points, original_points, original_points)
    iou = inter[:, 0, 0] / jnp.maximum(union[:, 0, 0], 1.0)
    return jnp.mean(iou)
